# Initial kernel scaffold; baseline (speedup 1.0000x reference)
import jax
import jax.numpy as jnp
from jax.experimental import pallas as pl


def kernel(x, edge_index, W1_l, W1_r, b1, W2_l, W2_r, b2):
    raise NotImplementedError("write your pallas kernel here")


# speedup vs baseline: 10.3828x; 10.3828x over previous
import functools

import jax
import jax.numpy as jnp
from jax import lax
from jax.experimental import pallas as pl
from jax.experimental.pallas import tpu as pltpu
from jax.experimental.pallas import tpu_sc as plsc

_N = 10000
_D = 128
_H = 16
_O = 16
_E = 320000

_LANES = 128
_EROWS = 2560
_RPT = _EROWS // 32
_NACC = 10112
_PER_TILE = _NACC // 16
_BN = 1000


def _make_sc_pass(width):
    mesh = plsc.VectorSubcoreMesh(core_axis_name="c", subcore_axis_name="s")

    def body(vals_hbm, src_hbm, dst_hbm, zeros_hbm, out_hbm,
             acc, src_t, dst_t, rows_t, sem):
        c = lax.axis_index("c")
        s = lax.axis_index("s")
        wid = s * 2 + c

        pltpu.sync_copy(zeros_hbm.at[pl.ds(s * _PER_TILE, _PER_TILE)],
                        acc.at[pl.ds(s * _PER_TILE, _PER_TILE)])
        pltpu.sync_copy(src_hbm.at[pl.ds(wid * _RPT, _RPT)], src_t)
        pltpu.sync_copy(dst_hbm.at[pl.ds(wid * _RPT, _RPT)], dst_t)
        plsc.subcore_barrier()

        def step(j, carry):
            pltpu.async_copy(vals_hbm.at[src_t.at[j]], rows_t, sem).wait()
            pltpu.sync_copy(rows_t, acc.at[dst_t.at[j]], add=True)
            return carry

        lax.fori_loop(0, _RPT, step, 0)
        plsc.subcore_barrier()

        pltpu.sync_copy(acc.at[pl.ds(s * _PER_TILE, _PER_TILE)],
                        out_hbm.at[c].at[pl.ds(s * _PER_TILE, _PER_TILE)])

    return pl.kernel(
        body,
        mesh=mesh,
        compiler_params=pltpu.CompilerParams(use_tc_tiling_on_sc=False),
        out_type=jax.ShapeDtypeStruct((2, _NACC, width), jnp.float32),
        scratch_types=[
            pltpu.VMEM_SHARED((_NACC, width), jnp.float32),
            pltpu.VMEM((_RPT, _LANES), jnp.int32),
            pltpu.VMEM((_RPT, _LANES), jnp.int32),
            pltpu.VMEM((_LANES, width), jnp.float32),
            pltpu.SemaphoreType.DMA,
        ],
    )


_sc_pass_cached = functools.lru_cache(maxsize=None)(_make_sc_pass)


def _tc1_body(x_ref, wpad_ref, wr_ref, vals_ref, z_ref):
    xb = x_ref[...]
    col = lax.broadcasted_iota(jnp.int32, (_BN, 32), 1)
    ones16 = jnp.where(col == 16, 1.0, 0.0).astype(jnp.float32)
    vals_ref[...] = jnp.dot(xb, wpad_ref[...],
                            preferred_element_type=jnp.float32) + ones16
    z_ref[...] = jnp.dot(xb, wr_ref[...], preferred_element_type=jnp.float32)


def _tc1(x, wpad, wr):
    return pl.pallas_call(
        _tc1_body,
        grid=(_N // _BN,),
        in_specs=[
            pl.BlockSpec((_BN, _D), lambda i: (i, 0)),
            pl.BlockSpec((_D, 32), lambda i: (0, 0)),
            pl.BlockSpec((_D, _H), lambda i: (0, 0)),
        ],
        out_specs=[
            pl.BlockSpec((_BN, 32), lambda i: (i, 0)),
            pl.BlockSpec((_BN, _H), lambda i: (i, 0)),
        ],
        out_shape=[
            jax.ShapeDtypeStruct((_N, 32), jnp.float32),
            jax.ShapeDtypeStruct((_N, _H), jnp.float32),
        ],
    )(x, wpad, wr)


def _tc2_body(p_ref, z1_ref, b1_ref, w2l_ref, w2r_ref, b2_ref,
              vals2_ref, z2_ref, inv_ref):
    sall = p_ref[0] + p_ref[1]
    ssum = sall[:, 0:16]
    cnt = sall[:, 16:17]
    inv = 1.0 / jnp.maximum(cnt, 1.0)
    h = jnp.maximum(ssum * inv + b1_ref[...] + z1_ref[...], 0.0)
    vals2_ref[...] = jnp.dot(h, w2l_ref[...], preferred_element_type=jnp.float32)
    z2_ref[...] = jnp.dot(h, w2r_ref[...],
                          preferred_element_type=jnp.float32) + b2_ref[...]
    inv_ref[...] = jnp.broadcast_to(inv, (_BN, _H))


def _tc2(parts1, z1, b1t, w2l, w2r, b2t):
    return pl.pallas_call(
        _tc2_body,
        grid=(_N // _BN,),
        in_specs=[
            pl.BlockSpec((2, _BN, 32), lambda i: (0, i, 0)),
            pl.BlockSpec((_BN, _H), lambda i: (i, 0)),
            pl.BlockSpec((_BN, _H), lambda i: (0, 0)),
            pl.BlockSpec((_H, _H), lambda i: (0, 0)),
            pl.BlockSpec((_H, _O), lambda i: (0, 0)),
            pl.BlockSpec((_BN, _O), lambda i: (0, 0)),
        ],
        out_specs=[
            pl.BlockSpec((_BN, _H), lambda i: (i, 0)),
            pl.BlockSpec((_BN, _O), lambda i: (i, 0)),
            pl.BlockSpec((_BN, _H), lambda i: (i, 0)),
        ],
        out_shape=[
            jax.ShapeDtypeStruct((_N, _H), jnp.float32),
            jax.ShapeDtypeStruct((_N, _O), jnp.float32),
            jax.ShapeDtypeStruct((_N, _H), jnp.float32),
        ],
    )(parts1, z1, b1t, w2l, w2r, b2t)


def _tc3_body(p_ref, inv_ref, z2_ref, out_ref):
    out_ref[...] = (p_ref[0] + p_ref[1]) * inv_ref[...] + z2_ref[...]


def _tc3(parts2, inv, z2):
    return pl.pallas_call(
        _tc3_body,
        grid=(_N // _BN,),
        in_specs=[
            pl.BlockSpec((2, _BN, _O), lambda i: (0, i, 0)),
            pl.BlockSpec((_BN, _O), lambda i: (i, 0)),
            pl.BlockSpec((_BN, _O), lambda i: (i, 0)),
        ],
        out_specs=pl.BlockSpec((_BN, _O), lambda i: (i, 0)),
        out_shape=jax.ShapeDtypeStruct((_N, _O), jnp.float32),
    )(parts2, inv, z2)


def kernel(x, edge_index, W1_l, W1_r, b1, W2_l, W2_r, b2):
    src = edge_index[0]
    dst = edge_index[1]
    pad = _EROWS * _LANES - _E
    src_p = jnp.concatenate(
        [src, jnp.zeros((pad,), jnp.int32)]).reshape(_EROWS, _LANES)
    dst_p = jnp.concatenate(
        [dst, jnp.full((pad,), _N, jnp.int32)]).reshape(_EROWS, _LANES)
    wpad = jnp.concatenate([W1_l, jnp.zeros((_D, _H), jnp.float32)], axis=1)
    zeros32 = jnp.zeros((_NACC, 32), jnp.float32)
    zeros16 = jnp.zeros((_NACC, _O), jnp.float32)
    b1t = jnp.broadcast_to(b1.reshape(1, _H), (_BN, _H))
    b2t = jnp.broadcast_to(b2.reshape(1, _O), (_BN, _O))

    vals1, z1 = _tc1(x, wpad, W1_r)
    parts1 = _sc_pass_cached(32)(vals1, src_p, dst_p, zeros32)
    vals2, z2, inv = _tc2(parts1, z1, b1t, W2_l, W2_r, b2t)
    parts2 = _sc_pass_cached(16)(vals2, src_p, dst_p, zeros16)
    return _tc3(parts2, inv, z2)


# speedup vs baseline: 12.4033x; 1.1946x over previous
import functools

import jax
import jax.numpy as jnp
from jax import lax
from jax.experimental import pallas as pl
from jax.experimental.pallas import tpu as pltpu
from jax.experimental.pallas import tpu_sc as plsc

_N = 10000
_D = 128
_H = 16
_O = 16
_E = 320000

_LANES = 128
_EROWS = 2560
_RPT = _EROWS // 32
_NACC = 10112
_PER_TILE = _NACC // 16
_BN = 1000
_K = 16


def _make_sc_pass(width):
    mesh = plsc.VectorSubcoreMesh(core_axis_name="c", subcore_axis_name="s")

    def body(vals_hbm, src_hbm, dst_hbm, zeros_hbm, out_hbm,
             acc, src_t, dst_t, rows_t, gsem, ssem):
        c = lax.axis_index("c")
        s = lax.axis_index("s")
        wid = s * 2 + c

        pltpu.sync_copy(zeros_hbm.at[pl.ds(s * _PER_TILE, _PER_TILE)],
                        acc.at[pl.ds(s * _PER_TILE, _PER_TILE)])
        pltpu.sync_copy(src_hbm.at[pl.ds(wid * _RPT, _RPT)], src_t)
        pltpu.sync_copy(dst_hbm.at[pl.ds(wid * _RPT, _RPT)], dst_t)
        plsc.subcore_barrier()

        def wave(w, carry):
            base = w * _K

            def fire_g(i, cc):
                pltpu.async_copy(vals_hbm.at[src_t.at[base + i]],
                                 rows_t.at[pl.ds(i * _LANES, _LANES)], gsem)
                return cc

            lax.fori_loop(0, _K, fire_g, 0)
            pltpu.make_async_copy(zeros_hbm.at[pl.ds(0, _K * _LANES)],
                                  rows_t, gsem).wait()

            def fire_s(i, cc):
                pltpu.async_copy(rows_t.at[pl.ds(i * _LANES, _LANES)],
                                 acc.at[dst_t.at[base + i]], ssem, add=True)
                return cc

            lax.fori_loop(0, _K, fire_s, 0)
            pltpu.make_async_copy(rows_t, acc.at[pl.ds(0, _K * _LANES)],
                                  ssem).wait()
            return carry

        lax.fori_loop(0, _RPT // _K, wave, 0)
        plsc.subcore_barrier()

        pltpu.sync_copy(acc.at[pl.ds(s * _PER_TILE, _PER_TILE)],
                        out_hbm.at[c].at[pl.ds(s * _PER_TILE, _PER_TILE)])

    return pl.kernel(
        body,
        mesh=mesh,
        compiler_params=pltpu.CompilerParams(use_tc_tiling_on_sc=False),
        out_type=jax.ShapeDtypeStruct((2, _NACC, width), jnp.float32),
        scratch_types=[
            pltpu.VMEM_SHARED((_NACC, width), jnp.float32),
            pltpu.VMEM((_RPT, _LANES), jnp.int32),
            pltpu.VMEM((_RPT, _LANES), jnp.int32),
            pltpu.VMEM((_K * _LANES, width), jnp.float32),
            pltpu.SemaphoreType.DMA,
            pltpu.SemaphoreType.DMA,
        ],
    )


_sc_pass_cached = functools.lru_cache(maxsize=None)(_make_sc_pass)


def _tc1_body(x_ref, wpad_ref, wr_ref, vals_ref, z_ref):
    xb = x_ref[...]
    col = lax.broadcasted_iota(jnp.int32, (_BN, 32), 1)
    ones16 = jnp.where(col == 16, 1.0, 0.0).astype(jnp.float32)
    vals_ref[...] = jnp.dot(xb, wpad_ref[...],
                            preferred_element_type=jnp.float32) + ones16
    z_ref[...] = jnp.dot(xb, wr_ref[...], preferred_element_type=jnp.float32)


def _tc1(x, wpad, wr):
    return pl.pallas_call(
        _tc1_body,
        grid=(_N // _BN,),
        in_specs=[
            pl.BlockSpec((_BN, _D), lambda i: (i, 0)),
            pl.BlockSpec((_D, 32), lambda i: (0, 0)),
            pl.BlockSpec((_D, _H), lambda i: (0, 0)),
        ],
        out_specs=[
            pl.BlockSpec((_BN, 32), lambda i: (i, 0)),
            pl.BlockSpec((_BN, _H), lambda i: (i, 0)),
        ],
        out_shape=[
            jax.ShapeDtypeStruct((_N, 32), jnp.float32),
            jax.ShapeDtypeStruct((_N, _H), jnp.float32),
        ],
    )(x, wpad, wr)


def _tc2_body(p_ref, z1_ref, b1_ref, w2l_ref, w2r_ref, b2_ref,
              vals2_ref, z2_ref, inv_ref):
    sall = p_ref[0] + p_ref[1]
    ssum = sall[:, 0:16]
    cnt = sall[:, 16:17]
    inv = 1.0 / jnp.maximum(cnt, 1.0)
    h = jnp.maximum(ssum * inv + b1_ref[...] + z1_ref[...], 0.0)
    vals2_ref[...] = jnp.dot(h, w2l_ref[...], preferred_element_type=jnp.float32)
    z2_ref[...] = jnp.dot(h, w2r_ref[...],
                          preferred_element_type=jnp.float32) + b2_ref[...]
    inv_ref[...] = jnp.broadcast_to(inv, (_BN, _H))


def _tc2(parts1, z1, b1t, w2l, w2r, b2t):
    return pl.pallas_call(
        _tc2_body,
        grid=(_N // _BN,),
        in_specs=[
            pl.BlockSpec((2, _BN, 32), lambda i: (0, i, 0)),
            pl.BlockSpec((_BN, _H), lambda i: (i, 0)),
            pl.BlockSpec((_BN, _H), lambda i: (0, 0)),
            pl.BlockSpec((_H, _H), lambda i: (0, 0)),
            pl.BlockSpec((_H, _O), lambda i: (0, 0)),
            pl.BlockSpec((_BN, _O), lambda i: (0, 0)),
        ],
        out_specs=[
            pl.BlockSpec((_BN, _H), lambda i: (i, 0)),
            pl.BlockSpec((_BN, _O), lambda i: (i, 0)),
            pl.BlockSpec((_BN, _H), lambda i: (i, 0)),
        ],
        out_shape=[
            jax.ShapeDtypeStruct((_N, _H), jnp.float32),
            jax.ShapeDtypeStruct((_N, _O), jnp.float32),
            jax.ShapeDtypeStruct((_N, _H), jnp.float32),
        ],
    )(parts1, z1, b1t, w2l, w2r, b2t)


def _tc3_body(p_ref, inv_ref, z2_ref, out_ref):
    out_ref[...] = (p_ref[0] + p_ref[1]) * inv_ref[...] + z2_ref[...]


def _tc3(parts2, inv, z2):
    return pl.pallas_call(
        _tc3_body,
        grid=(_N // _BN,),
        in_specs=[
            pl.BlockSpec((2, _BN, _O), lambda i: (0, i, 0)),
            pl.BlockSpec((_BN, _O), lambda i: (i, 0)),
            pl.BlockSpec((_BN, _O), lambda i: (i, 0)),
        ],
        out_specs=pl.BlockSpec((_BN, _O), lambda i: (i, 0)),
        out_shape=jax.ShapeDtypeStruct((_N, _O), jnp.float32),
    )(parts2, inv, z2)


def kernel(x, edge_index, W1_l, W1_r, b1, W2_l, W2_r, b2):
    src = edge_index[0]
    dst = edge_index[1]
    pad = _EROWS * _LANES - _E
    src_p = jnp.concatenate(
        [src, jnp.zeros((pad,), jnp.int32)]).reshape(_EROWS, _LANES)
    dst_p = jnp.concatenate(
        [dst, jnp.full((pad,), _N, jnp.int32)]).reshape(_EROWS, _LANES)
    wpad = jnp.concatenate([W1_l, jnp.zeros((_D, _H), jnp.float32)], axis=1)
    zeros32 = jnp.zeros((_NACC, 32), jnp.float32)
    zeros16 = jnp.zeros((_NACC, _O), jnp.float32)
    b1t = jnp.broadcast_to(b1.reshape(1, _H), (_BN, _H))
    b2t = jnp.broadcast_to(b2.reshape(1, _O), (_BN, _O))

    vals1, z1 = _tc1(x, wpad, W1_r)
    parts1 = _sc_pass_cached(32)(vals1, src_p, dst_p, zeros32)
    vals2, z2, inv = _tc2(parts1, z1, b1t, W2_l, W2_r, b2t)
    parts2 = _sc_pass_cached(16)(vals2, src_p, dst_p, zeros16)
    return _tc3(parts2, inv, z2)


# speedup vs baseline: 13.0381x; 1.0512x over previous
import functools

import jax
import jax.numpy as jnp
from jax import lax
from jax.experimental import pallas as pl
from jax.experimental.pallas import tpu as pltpu
from jax.experimental.pallas import tpu_sc as plsc

_N = 10000
_D = 128
_H = 16
_O = 16
_E = 320000

_LANES = 128
_EROWS = 2560
_RPT = _EROWS // 32
_NACC = 10112
_PER_TILE = _NACC // 16
_BN = 1000
_K = 10


def _make_sc_pass(width):
    mesh = plsc.VectorSubcoreMesh(core_axis_name="c", subcore_axis_name="s")

    def body(vals_hbm, src_hbm, dst_hbm, zeros_hbm, out_hbm,
             acc, src_t, dst_t, rows_t, gsem, gsem2, ssem, ssem2):
        c = lax.axis_index("c")
        s = lax.axis_index("s")
        wid = s * 2 + c

        pltpu.sync_copy(zeros_hbm.at[pl.ds(s * _PER_TILE, _PER_TILE)],
                        acc.at[pl.ds(s * _PER_TILE, _PER_TILE)])
        pltpu.sync_copy(src_hbm.at[pl.ds(wid * _RPT, _RPT)], src_t)
        pltpu.sync_copy(dst_hbm.at[pl.ds(wid * _RPT, _RPT)], dst_t)
        plsc.subcore_barrier()

        rows_a, rows_b = rows_t.at[0], rows_t.at[1]

        def fire_g(base, slab, sem):
            def f(i, cc):
                pltpu.async_copy(vals_hbm.at[src_t.at[base + i]],
                                 slab.at[pl.ds(i * _LANES, _LANES)], sem)
                return cc
            lax.fori_loop(0, _K, f, 0)

        def drain_g(slab, sem):
            pltpu.make_async_copy(zeros_hbm.at[pl.ds(0, _K * _LANES)],
                                  slab, sem).wait()

        def fire_s(base, slab, sem):
            def f(i, cc):
                pltpu.async_copy(slab.at[pl.ds(i * _LANES, _LANES)],
                                 acc.at[dst_t.at[base + i]], sem, add=True)
                return cc
            lax.fori_loop(0, _K, f, 0)

        def drain_s(slab, sem):
            pltpu.make_async_copy(slab, acc.at[pl.ds(0, _K * _LANES)],
                                  sem).wait()

        n_pairs = _RPT // (2 * _K)
        fire_g(0, rows_a, gsem)

        def pair(p, carry):
            b0 = (2 * p) * _K
            b1 = b0 + _K
            drain_g(rows_a, gsem)
            fire_g(b1, rows_b, gsem2)
            fire_s(b0, rows_a, ssem)
            drain_s(rows_a, ssem)
            drain_g(rows_b, gsem2)

            @pl.when(p + 1 < n_pairs)
            def _():
                fire_g(b1 + _K, rows_a, gsem)

            fire_s(b1, rows_b, ssem2)
            drain_s(rows_b, ssem2)
            return carry

        lax.fori_loop(0, n_pairs, pair, 0)
        plsc.subcore_barrier()

        pltpu.sync_copy(acc.at[pl.ds(s * _PER_TILE, _PER_TILE)],
                        out_hbm.at[c].at[pl.ds(s * _PER_TILE, _PER_TILE)])

    return pl.kernel(
        body,
        mesh=mesh,
        compiler_params=pltpu.CompilerParams(use_tc_tiling_on_sc=False),
        out_type=jax.ShapeDtypeStruct((2, _NACC, width), jnp.float32),
        scratch_types=[
            pltpu.VMEM_SHARED((_NACC, width), jnp.float32),
            pltpu.VMEM((_RPT, _LANES), jnp.int32),
            pltpu.VMEM((_RPT, _LANES), jnp.int32),
            pltpu.VMEM((2, _K * _LANES, width), jnp.float32),
            pltpu.SemaphoreType.DMA,
            pltpu.SemaphoreType.DMA,
            pltpu.SemaphoreType.DMA,
            pltpu.SemaphoreType.DMA,
        ],
    )


_sc_pass_cached = functools.lru_cache(maxsize=None)(_make_sc_pass)


def _tc1_body(x_ref, wpad_ref, wr_ref, vals_ref, z_ref):
    xb = x_ref[...]
    col = lax.broadcasted_iota(jnp.int32, (_BN, 32), 1)
    ones16 = jnp.where(col == 16, 1.0, 0.0).astype(jnp.float32)
    vals_ref[...] = jnp.dot(xb, wpad_ref[...],
                            preferred_element_type=jnp.float32) + ones16
    z_ref[...] = jnp.dot(xb, wr_ref[...], preferred_element_type=jnp.float32)


def _tc1(x, wpad, wr):
    return pl.pallas_call(
        _tc1_body,
        grid=(_N // _BN,),
        in_specs=[
            pl.BlockSpec((_BN, _D), lambda i: (i, 0)),
            pl.BlockSpec((_D, 32), lambda i: (0, 0)),
            pl.BlockSpec((_D, _H), lambda i: (0, 0)),
        ],
        out_specs=[
            pl.BlockSpec((_BN, 32), lambda i: (i, 0)),
            pl.BlockSpec((_BN, _H), lambda i: (i, 0)),
        ],
        out_shape=[
            jax.ShapeDtypeStruct((_N, 32), jnp.float32),
            jax.ShapeDtypeStruct((_N, _H), jnp.float32),
        ],
    )(x, wpad, wr)


def _tc2_body(p_ref, z1_ref, b1_ref, w2l_ref, w2r_ref, b2_ref,
              vals2_ref, z2_ref, inv_ref):
    sall = p_ref[0] + p_ref[1]
    ssum = sall[:, 0:16]
    cnt = sall[:, 16:17]
    inv = 1.0 / jnp.maximum(cnt, 1.0)
    h = jnp.maximum(ssum * inv + b1_ref[...] + z1_ref[...], 0.0)
    vals2_ref[...] = jnp.dot(h, w2l_ref[...], preferred_element_type=jnp.float32)
    z2_ref[...] = jnp.dot(h, w2r_ref[...],
                          preferred_element_type=jnp.float32) + b2_ref[...]
    inv_ref[...] = jnp.broadcast_to(inv, (_BN, _H))


def _tc2(parts1, z1, b1t, w2l, w2r, b2t):
    return pl.pallas_call(
        _tc2_body,
        grid=(_N // _BN,),
        in_specs=[
            pl.BlockSpec((2, _BN, 32), lambda i: (0, i, 0)),
            pl.BlockSpec((_BN, _H), lambda i: (i, 0)),
            pl.BlockSpec((_BN, _H), lambda i: (0, 0)),
            pl.BlockSpec((_H, _H), lambda i: (0, 0)),
            pl.BlockSpec((_H, _O), lambda i: (0, 0)),
            pl.BlockSpec((_BN, _O), lambda i: (0, 0)),
        ],
        out_specs=[
            pl.BlockSpec((_BN, _H), lambda i: (i, 0)),
            pl.BlockSpec((_BN, _O), lambda i: (i, 0)),
            pl.BlockSpec((_BN, _H), lambda i: (i, 0)),
        ],
        out_shape=[
            jax.ShapeDtypeStruct((_N, _H), jnp.float32),
            jax.ShapeDtypeStruct((_N, _O), jnp.float32),
            jax.ShapeDtypeStruct((_N, _H), jnp.float32),
        ],
    )(parts1, z1, b1t, w2l, w2r, b2t)


def _tc3_body(p_ref, inv_ref, z2_ref, out_ref):
    out_ref[...] = (p_ref[0] + p_ref[1]) * inv_ref[...] + z2_ref[...]


def _tc3(parts2, inv, z2):
    return pl.pallas_call(
        _tc3_body,
        grid=(_N // _BN,),
        in_specs=[
            pl.BlockSpec((2, _BN, _O), lambda i: (0, i, 0)),
            pl.BlockSpec((_BN, _O), lambda i: (i, 0)),
            pl.BlockSpec((_BN, _O), lambda i: (i, 0)),
        ],
        out_specs=pl.BlockSpec((_BN, _O), lambda i: (i, 0)),
        out_shape=jax.ShapeDtypeStruct((_N, _O), jnp.float32),
    )(parts2, inv, z2)


def kernel(x, edge_index, W1_l, W1_r, b1, W2_l, W2_r, b2):
    src = edge_index[0]
    dst = edge_index[1]
    pad = _EROWS * _LANES - _E
    src_p = jnp.concatenate(
        [src, jnp.zeros((pad,), jnp.int32)]).reshape(_EROWS, _LANES)
    dst_p = jnp.concatenate(
        [dst, jnp.full((pad,), _N, jnp.int32)]).reshape(_EROWS, _LANES)
    wpad = jnp.concatenate([W1_l, jnp.zeros((_D, _H), jnp.float32)], axis=1)
    zeros32 = jnp.zeros((_NACC, 32), jnp.float32)
    zeros16 = jnp.zeros((_NACC, _O), jnp.float32)
    b1t = jnp.broadcast_to(b1.reshape(1, _H), (_BN, _H))
    b2t = jnp.broadcast_to(b2.reshape(1, _O), (_BN, _O))

    vals1, z1 = _tc1(x, wpad, W1_r)
    parts1 = _sc_pass_cached(32)(vals1, src_p, dst_p, zeros32)
    vals2, z2, inv = _tc2(parts1, z1, b1t, W2_l, W2_r, b2t)
    parts2 = _sc_pass_cached(16)(vals2, src_p, dst_p, zeros16)
    return _tc3(parts2, inv, z2)


# speedup vs baseline: 15.0921x; 1.1575x over previous
import functools

import jax
import jax.numpy as jnp
from jax import lax
from jax.experimental import pallas as pl
from jax.experimental.pallas import tpu as pltpu
from jax.experimental.pallas import tpu_sc as plsc

_N = 10000
_D = 128
_H = 16
_O = 16
_E = 320000

_LANES = 128
_EROWS = 2560
_RPT = _EROWS // 32
_NACC = 10112
_PER_TILE = _NACC // 16
_BN = 1000
_K = 8


def _make_sc_pass(with_counts):
    mesh = plsc.VectorSubcoreMesh(core_axis_name="c", subcore_axis_name="s")

    def body(vals_hbm, src_hbm, dst_hbm, zeros_hbm, ones_hbm, *rest):
        if with_counts:
            (out_hbm, outc_hbm, acc, acc_cnt, src_t, dst_t, slab, ones_t,
             gsem, ssem, osem) = rest
        else:
            (out_hbm, acc, src_t, dst_t, slab, gsem, ssem) = rest
        c = lax.axis_index("c")
        s = lax.axis_index("s")
        wid = s * 2 + c

        pltpu.sync_copy(zeros_hbm.at[pl.ds(s * _PER_TILE, _PER_TILE)],
                        acc.at[pl.ds(s * _PER_TILE, _PER_TILE)])
        if with_counts:
            pltpu.sync_copy(zeros_hbm.at[pl.ds(s * _PER_TILE, _PER_TILE)],
                            acc_cnt.at[pl.ds(s * _PER_TILE, _PER_TILE)])
            pltpu.sync_copy(ones_hbm, ones_t)
        pltpu.sync_copy(src_hbm.at[pl.ds(wid * _RPT, _RPT)], src_t)
        pltpu.sync_copy(dst_hbm.at[pl.ds(wid * _RPT, _RPT)], dst_t)
        plsc.subcore_barrier()

        def step(it, carry):
            base = it * _K
            hg = [pltpu.async_copy(vals_hbm.at[src_t.at[base + j]],
                                   slab.at[pl.ds(j * _LANES, _LANES)], gsem)
                  for j in range(_K)]
            hc = []
            if with_counts:
                hc = [pltpu.async_copy(ones_t, acc_cnt.at[dst_t.at[base + j]],
                                       osem, add=True)
                      for j in range(_K)]
            for h in hg:
                h.wait()
            hs = [pltpu.async_copy(slab.at[pl.ds(j * _LANES, _LANES)],
                                   acc.at[dst_t.at[base + j]], ssem, add=True)
                  for j in range(_K)]
            for h in hs:
                h.wait()
            for h in hc:
                h.wait()
            return carry

        lax.fori_loop(0, _RPT // _K, step, 0)
        plsc.subcore_barrier()

        pltpu.sync_copy(acc.at[pl.ds(s * _PER_TILE, _PER_TILE)],
                        out_hbm.at[c].at[pl.ds(s * _PER_TILE, _PER_TILE)])
        if with_counts:
            pltpu.sync_copy(acc_cnt.at[pl.ds(s * _PER_TILE, _PER_TILE)],
                            outc_hbm.at[c].at[pl.ds(s * _PER_TILE, _PER_TILE)])

    out_type = [jax.ShapeDtypeStruct((2, _NACC, _H), jnp.float32)]
    scratch = [pltpu.VMEM_SHARED((_NACC, _H), jnp.float32)]
    if with_counts:
        out_type.append(jax.ShapeDtypeStruct((2, _NACC, _H), jnp.float32))
        scratch.append(pltpu.VMEM_SHARED((_NACC, _H), jnp.float32))
    scratch += [
        pltpu.VMEM((_RPT, _LANES), jnp.int32),
        pltpu.VMEM((_RPT, _LANES), jnp.int32),
        pltpu.VMEM((_K * _LANES, _H), jnp.float32),
    ]
    if with_counts:
        scratch.append(pltpu.VMEM((_LANES, _H), jnp.float32))
    scratch += [pltpu.SemaphoreType.DMA] * (3 if with_counts else 2)

    return pl.kernel(
        body,
        mesh=mesh,
        compiler_params=pltpu.CompilerParams(use_tc_tiling_on_sc=False),
        out_type=out_type,
        scratch_types=scratch,
    )


_sc_pass_cached = functools.lru_cache(maxsize=None)(_make_sc_pass)


def _tc1_body(x_ref, wl_ref, wr_ref, vals_ref, z_ref):
    xb = x_ref[...]
    vals_ref[...] = jnp.dot(xb, wl_ref[...], preferred_element_type=jnp.float32)
    z_ref[...] = jnp.dot(xb, wr_ref[...], preferred_element_type=jnp.float32)


def _tc1(x, wl, wr):
    return pl.pallas_call(
        _tc1_body,
        grid=(_N // _BN,),
        in_specs=[
            pl.BlockSpec((_BN, _D), lambda i: (i, 0)),
            pl.BlockSpec((_D, _H), lambda i: (0, 0)),
            pl.BlockSpec((_D, _H), lambda i: (0, 0)),
        ],
        out_specs=[
            pl.BlockSpec((_BN, _H), lambda i: (i, 0)),
            pl.BlockSpec((_BN, _H), lambda i: (i, 0)),
        ],
        out_shape=[
            jax.ShapeDtypeStruct((_N, _H), jnp.float32),
            jax.ShapeDtypeStruct((_N, _H), jnp.float32),
        ],
    )(x, wl, wr)


def _tc2_body(p_ref, pc_ref, z1_ref, b1_ref, w2l_ref, w2r_ref, b2_ref,
              vals2_ref, z2_ref, inv_ref):
    ssum = p_ref[0] + p_ref[1]
    cnt = pc_ref[0] + pc_ref[1]
    inv = 1.0 / jnp.maximum(cnt, 1.0)
    h = jnp.maximum(ssum * inv + b1_ref[...] + z1_ref[...], 0.0)
    vals2_ref[...] = jnp.dot(h, w2l_ref[...], preferred_element_type=jnp.float32)
    z2_ref[...] = jnp.dot(h, w2r_ref[...],
                          preferred_element_type=jnp.float32) + b2_ref[...]
    inv_ref[...] = inv


def _tc2(parts1, parts1c, z1, b1t, w2l, w2r, b2t):
    return pl.pallas_call(
        _tc2_body,
        grid=(_N // _BN,),
        in_specs=[
            pl.BlockSpec((2, _BN, _H), lambda i: (0, i, 0)),
            pl.BlockSpec((2, _BN, _H), lambda i: (0, i, 0)),
            pl.BlockSpec((_BN, _H), lambda i: (i, 0)),
            pl.BlockSpec((_BN, _H), lambda i: (0, 0)),
            pl.BlockSpec((_H, _H), lambda i: (0, 0)),
            pl.BlockSpec((_H, _O), lambda i: (0, 0)),
            pl.BlockSpec((_BN, _O), lambda i: (0, 0)),
        ],
        out_specs=[
            pl.BlockSpec((_BN, _H), lambda i: (i, 0)),
            pl.BlockSpec((_BN, _O), lambda i: (i, 0)),
            pl.BlockSpec((_BN, _H), lambda i: (i, 0)),
        ],
        out_shape=[
            jax.ShapeDtypeStruct((_N, _H), jnp.float32),
            jax.ShapeDtypeStruct((_N, _O), jnp.float32),
            jax.ShapeDtypeStruct((_N, _H), jnp.float32),
        ],
    )(parts1, parts1c, z1, b1t, w2l, w2r, b2t)


def _tc3_body(p_ref, inv_ref, z2_ref, out_ref):
    out_ref[...] = (p_ref[0] + p_ref[1]) * inv_ref[...] + z2_ref[...]


def _tc3(parts2, inv, z2):
    return pl.pallas_call(
        _tc3_body,
        grid=(_N // _BN,),
        in_specs=[
            pl.BlockSpec((2, _BN, _O), lambda i: (0, i, 0)),
            pl.BlockSpec((_BN, _O), lambda i: (i, 0)),
            pl.BlockSpec((_BN, _O), lambda i: (i, 0)),
        ],
        out_specs=pl.BlockSpec((_BN, _O), lambda i: (i, 0)),
        out_shape=jax.ShapeDtypeStruct((_N, _O), jnp.float32),
    )(parts2, inv, z2)


def kernel(x, edge_index, W1_l, W1_r, b1, W2_l, W2_r, b2):
    src = edge_index[0]
    dst = edge_index[1]
    pad = _EROWS * _LANES - _E
    src_p = jnp.concatenate(
        [src, jnp.zeros((pad,), jnp.int32)]).reshape(_EROWS, _LANES)
    dst_p = jnp.concatenate(
        [dst, jnp.full((pad,), _N, jnp.int32)]).reshape(_EROWS, _LANES)
    zeros16 = jnp.zeros((_NACC, _H), jnp.float32)
    ones16 = jnp.ones((_LANES, _H), jnp.float32)
    b1t = jnp.broadcast_to(b1.reshape(1, _H), (_BN, _H))
    b2t = jnp.broadcast_to(b2.reshape(1, _O), (_BN, _O))

    vals1, z1 = _tc1(x, W1_l, W1_r)
    parts1, parts1c = _sc_pass_cached(True)(vals1, src_p, dst_p, zeros16, ones16)
    vals2, z2, inv = _tc2(parts1, parts1c, z1, b1t, W2_l, W2_r, b2t)
    (parts2,) = _sc_pass_cached(False)(vals2, src_p, dst_p, zeros16, ones16)
    return _tc3(parts2, inv, z2)
